# baseline (speedup 1.0000x reference)
import functools

import jax
import jax.numpy as jnp
from jax import lax
from jax.experimental import pallas as pl
from jax.experimental.pallas import tpu as pltpu
from jax.experimental.pallas import tpu_sc as plsc

N_USERS = 50000
N_ITEMS = 50000
N_NODES = N_USERS + N_ITEMS
N_EDGES = 3200000
D = 16
NS = 16
NC = 2
NW = NC * NS
C = 512
SUB = 128
NSUB = C // SUB
NCH_C0 = 210
NCH_C1 = 182
N_CHUNKS = N_EDGES // C
FULL_ZBLK = N_NODES // C
TAIL_ROWS = N_NODES - FULL_ZBLK * C
SEL_T = 2048 // NW

_MESH = plsc.VectorSubcoreMesh(core_axis_name="c", subcore_axis_name="s",
                               num_cores=NC)
_SC_PARAMS = pltpu.CompilerParams(needs_layout_passes=False,
                                  use_tc_tiling_on_sc=False)


def _layer_body(tbl, esrc, edst2, ew, pa, pb,
                acc, src0, src1, w0, w1, dst0, dst1, rows0, rows1,
                esem, dsem, gsem, ssem):
    cid = lax.axis_index("c")
    wid = lax.axis_index("s")
    w32 = cid * NS + wid
    tile0 = jnp.where(cid == 0, wid * NCH_C0,
                      NS * NCH_C0 + wid * NCH_C1)
    nch = jnp.where(cid == 0, NCH_C0, NCH_C1)

    def gather_issue(src_v, rows_v):
        for s in range(NSUB):
            pltpu.async_copy(tbl.at[src_v.at[pl.ds(s * SUB, SUB)]],
                             rows_v.at[pl.ds(s * SUB, SUB)], gsem)

    def gather_wait(src_v, rows_v):
        pltpu.make_async_copy(tbl.at[pl.ds(0, C)], rows_v, gsem).wait()

    def scatter_issue(rows_v, dst_v):
        for s in range(NSUB):
            pltpu.async_copy(rows_v.at[pl.ds(s * SUB, SUB)],
                             acc.at[dst_v.at[s]], ssem, add=True)

    def scatter_wait(rows_v, dst_v):
        pltpu.make_async_copy(rows_v, acc.at[pl.ds(0, C)], ssem).wait()

    def srcw_issue(c, src_v, w_v):
        cc = jnp.minimum(c, N_CHUNKS - 1)
        pltpu.async_copy(esrc.at[pl.ds(cc * C, C)], src_v, esem)
        pltpu.async_copy(ew.at[pl.ds(cc * C, C)], w_v, esem)

    def srcw_wait(c, src_v, w_v):
        cc = jnp.minimum(c, N_CHUNKS - 1)
        pltpu.make_async_copy(esrc.at[pl.ds(cc * C, C)], src_v, esem).wait()
        pltpu.make_async_copy(ew.at[pl.ds(cc * C, C)], w_v, esem).wait()
        @pl.when(c >= N_CHUNKS)
        def _():
            @pl.loop(0, C // D)
            def _zw(g):
                w_v[pl.ds(g * D, D)] = jnp.zeros((D,), jnp.float32)

    def dst_issue(c, dst_v):
        cc = jnp.minimum(c, N_CHUNKS - 1)
        pltpu.async_copy(edst2.at[pl.ds(cc * NSUB, NSUB)], dst_v, dsem)

    def dst_wait(c, dst_v):
        cc = jnp.minimum(c, N_CHUNKS - 1)
        pltpu.make_async_copy(edst2.at[pl.ds(cc * NSUB, NSUB)], dst_v,
                              dsem).wait()

    def compute(rows_v, w_v):
        @pl.loop(0, C // D)
        def _g(g):
            w16 = w_v[pl.ds(g * D, D)]
            for i in range(D):
                bw = jnp.take_along_axis(
                    w16, jnp.full((D,), i, jnp.int32), axis=0,
                    mode="promise_in_bounds")
                j = g * D + i
                rows_v[j] = rows_v[j] * bw

    @pl.loop(0, C)
    def _zb(j):
        rows0[j] = jnp.zeros((D,), jnp.float32)

    for k in range(FULL_ZBLK // NS + 1):
        b = wid + NS * k

        @pl.when(b < FULL_ZBLK)
        def _():
            pltpu.sync_copy(rows0, acc.at[pl.ds(b * C, C)])

        @pl.when(b == FULL_ZBLK)
        def _():
            pltpu.sync_copy(rows0.at[pl.ds(0, TAIL_ROWS)],
                            acc.at[pl.ds(FULL_ZBLK * C, TAIL_ROWS)])
    plsc.subcore_barrier()

    pltpu.sync_copy(esrc.at[pl.ds(tile0 * C, C)], src0)
    pltpu.sync_copy(ew.at[pl.ds(tile0 * C, C)], w0)
    pltpu.sync_copy(edst2.at[pl.ds(tile0 * NSUB, NSUB)], dst0)
    gather_issue(src0, rows0)
    srcw_issue(tile0 + 1, src1, w1)

    @pl.loop(0, nch, step=2)
    def _pair(k2):
        c = tile0 + k2
        gather_wait(src0, rows0)

        @pl.when(k2 > 0)
        def _():
            scatter_wait(rows1, dst1)

        dst_issue(c + 1, dst1)
        srcw_wait(c + 1, src1, w1)
        gather_issue(src1, rows1)

        @pl.when(k2 > 0)
        def _():
            dst_wait(c, dst0)

        compute(rows0, w0)
        scatter_issue(rows0, dst0)

        @pl.when(k2 < nch - 2)
        def _():
            srcw_issue(c + 2, src0, w0)

        gather_wait(src1, rows1)
        scatter_wait(rows0, dst0)

        @pl.when(k2 < nch - 2)
        def _():
            dst_issue(c + 2, dst0)
            srcw_wait(c + 2, src0, w0)
            gather_issue(src0, rows0)

        dst_wait(c + 1, dst1)
        compute(rows1, w1)
        scatter_issue(rows1, dst1)

        @pl.when(k2 < nch - 2)
        def _():
            srcw_issue(c + 3, src1, w1)

    scatter_wait(rows1, dst1)
    plsc.subcore_barrier()

    for k in range(FULL_ZBLK // NS + 1):
        b = wid + NS * k
        for the_cid, out_t in ((0, pa), (1, pb)):

            @pl.when(jnp.logical_and(cid == the_cid, b < FULL_ZBLK))
            def _():
                pltpu.sync_copy(acc.at[pl.ds(b * C, C)],
                                out_t.at[pl.ds(b * C, C)])

            @pl.when(jnp.logical_and(cid == the_cid, b == FULL_ZBLK))
            def _():
                pltpu.sync_copy(acc.at[pl.ds(FULL_ZBLK * C, TAIL_ROWS)],
                                out_t.at[pl.ds(FULL_ZBLK * C, TAIL_ROWS)])


_layer = functools.partial(
    pl.kernel,
    out_type=[
        jax.ShapeDtypeStruct((N_NODES, D), jnp.float32),
        jax.ShapeDtypeStruct((N_NODES, D), jnp.float32),
    ],
    mesh=_MESH,
    compiler_params=_SC_PARAMS,
    scratch_types=[
        pltpu.VMEM_SHARED((N_NODES, D), jnp.float32),
        pltpu.VMEM((C,), jnp.int32),
        pltpu.VMEM((C,), jnp.int32),
        pltpu.VMEM((C,), jnp.float32),
        pltpu.VMEM((C,), jnp.float32),
        pltpu.VMEM((NSUB, SUB), jnp.int32),
        pltpu.VMEM((NSUB, SUB), jnp.int32),
        pltpu.VMEM((C, D), jnp.float32),
        pltpu.VMEM((C, D), jnp.float32),
        pltpu.SemaphoreType.DMA,
        pltpu.SemaphoreType.DMA,
        pltpu.SemaphoreType.DMA,
        pltpu.SemaphoreType.DMA,
    ],
)(_layer_body)


def _epi_body(em0, e1, e2, p3a, p3b, users, items, selo, idx, buf, gsem):
    cid = lax.axis_index("c")
    wid = lax.axis_index("s")
    w32 = cid * NS + wid

    @pl.loop(0, SEL_T)
    def _zb(j):
        buf[j] = jnp.zeros((D,), jnp.float32)

    @pl.when(w32 < NS)
    def _():
        pltpu.sync_copy(users.at[pl.ds(w32 * SEL_T, SEL_T)], idx)

    @pl.when(w32 >= NS)
    def _():
        pltpu.sync_copy(items.at[pl.ds((w32 - NS) * SEL_T, SEL_T)], idx)

    off = jnp.where(w32 < NS, 0, N_USERS).astype(jnp.int32)
    for g in range(SEL_T // D):
        idx[pl.ds(g * D, D)] = idx[pl.ds(g * D, D)] + off

    gds = [pltpu.async_copy(t.at[idx], buf, gsem, add=True)
           for t in (em0, e1, e2, p3a, p3b)]
    for dsc in gds:
        dsc.wait()

    @pl.loop(0, SEL_T)
    def _mean(r):
        buf[r] = buf[r] * 0.25

    pltpu.sync_copy(buf, selo.at[pl.ds(w32 * SEL_T, SEL_T)])


_epi = functools.partial(
    pl.kernel,
    out_type=[jax.ShapeDtypeStruct((2048, D), jnp.float32)],
    mesh=_MESH,
    compiler_params=_SC_PARAMS,
    scratch_types=[
        pltpu.VMEM((SEL_T,), jnp.int32),
        pltpu.VMEM((SEL_T, D), jnp.float32),
        pltpu.SemaphoreType.DMA,
    ],
)(_epi_body)


def _combine_body(a_ref, b_ref, o_ref):
    o_ref[...] = a_ref[...] + b_ref[...]


_combine = pl.pallas_call(
    _combine_body,
    out_shape=jax.ShapeDtypeStruct((N_NODES // D, D * D), jnp.float32),
    compiler_params=pltpu.CompilerParams(
        vmem_limit_bytes=100 * 1024 * 1024),
)


def _ratings_body(u_ref, it_ref, out_ref):
    out_ref[...] = jax.nn.sigmoid(
        jnp.dot(u_ref[...], it_ref[...], preferred_element_type=jnp.float32))


_ratings = pl.pallas_call(
    _ratings_body,
    out_shape=jax.ShapeDtypeStruct((1024, 1024), jnp.float32),
)


def _comb(pa, pb):
    e = _combine(pa.reshape(N_NODES // D, D * D),
                 pb.reshape(N_NODES // D, D * D))
    return e.reshape(N_NODES, D)


def kernel(user_table, item_table, edge_weight, edge_src, edge_dst, users, items):
    em0 = jnp.concatenate([user_table, item_table], axis=0)
    edst2 = edge_dst.reshape(N_EDGES // SUB, SUB)

    p1a, p1b = _layer(em0, edge_src, edst2, edge_weight)
    e1 = _comb(p1a, p1b)
    p2a, p2b = _layer(e1, edge_src, edst2, edge_weight)
    e2 = _comb(p2a, p2b)
    p3a, p3b = _layer(e2, edge_src, edst2, edge_weight)
    (sel,) = _epi(em0, e1, e2, p3a, p3b, users, items)
    u = sel[:1024]
    it_t = sel[1024:].T
    return _ratings(u, it_t)


# speedup vs baseline: 1.0566x; 1.0566x over previous
import functools

import jax
import jax.numpy as jnp
from jax import lax
from jax.experimental import pallas as pl
from jax.experimental.pallas import tpu as pltpu
from jax.experimental.pallas import tpu_sc as plsc

N_USERS = 50000
N_ITEMS = 50000
N_NODES = N_USERS + N_ITEMS
N_EDGES = 3200000
D = 16
NS = 16
NC = 2
NW = NC * NS
C = 512
SUB = 128
NSUB = C // SUB
NCH_C0 = 196
NCH_C1 = 196
N_CHUNKS = N_EDGES // C
FULL_ZBLK = N_NODES // C
TAIL_ROWS = N_NODES - FULL_ZBLK * C
SEL_T = 2048 // NW

_MESH = plsc.VectorSubcoreMesh(core_axis_name="c", subcore_axis_name="s",
                               num_cores=NC)
_SC_PARAMS = pltpu.CompilerParams(needs_layout_passes=False,
                                  use_tc_tiling_on_sc=False)


def _layer_body(tbl, esrc, edst2, ew, pa, pb,
                acc, src0, src1, w0, w1, dst0, dst1, rows0, rows1,
                esem, dsem, gsem, ssem):
    cid = lax.axis_index("c")
    wid = lax.axis_index("s")
    w32 = cid * NS + wid
    tile0 = jnp.where(cid == 0, wid * NCH_C0,
                      NS * NCH_C0 + wid * NCH_C1)
    nch = jnp.where(cid == 0, NCH_C0, NCH_C1)

    def gather_issue(src_v, rows_v):
        for s in range(NSUB):
            pltpu.async_copy(tbl.at[src_v.at[pl.ds(s * SUB, SUB)]],
                             rows_v.at[pl.ds(s * SUB, SUB)], gsem)

    def gather_wait(src_v, rows_v):
        pltpu.make_async_copy(tbl.at[pl.ds(0, C)], rows_v, gsem).wait()

    def scatter_issue(rows_v, dst_v):
        for s in range(NSUB):
            pltpu.async_copy(rows_v.at[pl.ds(s * SUB, SUB)],
                             acc.at[dst_v.at[s]], ssem, add=True)

    def scatter_wait(rows_v, dst_v):
        pltpu.make_async_copy(rows_v, acc.at[pl.ds(0, C)], ssem).wait()

    def srcw_issue(c, src_v, w_v):
        cc = jnp.minimum(c, N_CHUNKS - 1)
        pltpu.async_copy(esrc.at[pl.ds(cc * C, C)], src_v, esem)
        pltpu.async_copy(ew.at[pl.ds(cc * C, C)], w_v, esem)

    def srcw_wait(c, src_v, w_v):
        cc = jnp.minimum(c, N_CHUNKS - 1)
        pltpu.make_async_copy(esrc.at[pl.ds(cc * C, C)], src_v, esem).wait()
        pltpu.make_async_copy(ew.at[pl.ds(cc * C, C)], w_v, esem).wait()
        @pl.when(c >= N_CHUNKS)
        def _():
            @pl.loop(0, C // D)
            def _zw(g):
                w_v[pl.ds(g * D, D)] = jnp.zeros((D,), jnp.float32)

    def dst_issue(c, dst_v):
        cc = jnp.minimum(c, N_CHUNKS - 1)
        pltpu.async_copy(edst2.at[pl.ds(cc * NSUB, NSUB)], dst_v, dsem)

    def dst_wait(c, dst_v):
        cc = jnp.minimum(c, N_CHUNKS - 1)
        pltpu.make_async_copy(edst2.at[pl.ds(cc * NSUB, NSUB)], dst_v,
                              dsem).wait()

    def compute(rows_v, w_v):
        @pl.loop(0, C // D)
        def _g(g):
            w16 = w_v[pl.ds(g * D, D)]
            for i in range(D):
                bw = jnp.take_along_axis(
                    w16, jnp.full((D,), i, jnp.int32), axis=0,
                    mode="promise_in_bounds")
                j = g * D + i
                rows_v[j] = rows_v[j] * bw

    @pl.loop(0, C)
    def _zb(j):
        rows0[j] = jnp.zeros((D,), jnp.float32)

    for k in range(FULL_ZBLK // NS + 1):
        b = wid + NS * k

        @pl.when(b < FULL_ZBLK)
        def _():
            pltpu.sync_copy(rows0, acc.at[pl.ds(b * C, C)])

        @pl.when(b == FULL_ZBLK)
        def _():
            pltpu.sync_copy(rows0.at[pl.ds(0, TAIL_ROWS)],
                            acc.at[pl.ds(FULL_ZBLK * C, TAIL_ROWS)])
    plsc.subcore_barrier()

    pltpu.sync_copy(esrc.at[pl.ds(tile0 * C, C)], src0)
    pltpu.sync_copy(ew.at[pl.ds(tile0 * C, C)], w0)
    pltpu.sync_copy(edst2.at[pl.ds(tile0 * NSUB, NSUB)], dst0)
    gather_issue(src0, rows0)
    srcw_issue(tile0 + 1, src1, w1)

    @pl.loop(0, nch, step=2)
    def _pair(k2):
        c = tile0 + k2
        gather_wait(src0, rows0)

        @pl.when(k2 > 0)
        def _():
            scatter_wait(rows1, dst1)

        dst_issue(c + 1, dst1)
        srcw_wait(c + 1, src1, w1)
        gather_issue(src1, rows1)

        @pl.when(k2 > 0)
        def _():
            dst_wait(c, dst0)

        compute(rows0, w0)
        scatter_issue(rows0, dst0)

        @pl.when(k2 < nch - 2)
        def _():
            srcw_issue(c + 2, src0, w0)

        gather_wait(src1, rows1)
        scatter_wait(rows0, dst0)

        @pl.when(k2 < nch - 2)
        def _():
            dst_issue(c + 2, dst0)
            srcw_wait(c + 2, src0, w0)
            gather_issue(src0, rows0)

        dst_wait(c + 1, dst1)
        compute(rows1, w1)
        scatter_issue(rows1, dst1)

        @pl.when(k2 < nch - 2)
        def _():
            srcw_issue(c + 3, src1, w1)

    scatter_wait(rows1, dst1)
    plsc.subcore_barrier()

    for k in range(FULL_ZBLK // NS + 1):
        b = wid + NS * k
        for the_cid, out_t in ((0, pa), (1, pb)):

            @pl.when(jnp.logical_and(cid == the_cid, b < FULL_ZBLK))
            def _():
                pltpu.sync_copy(acc.at[pl.ds(b * C, C)],
                                out_t.at[pl.ds(b * C, C)])

            @pl.when(jnp.logical_and(cid == the_cid, b == FULL_ZBLK))
            def _():
                pltpu.sync_copy(acc.at[pl.ds(FULL_ZBLK * C, TAIL_ROWS)],
                                out_t.at[pl.ds(FULL_ZBLK * C, TAIL_ROWS)])


_layer = functools.partial(
    pl.kernel,
    out_type=[
        jax.ShapeDtypeStruct((N_NODES, D), jnp.float32),
        jax.ShapeDtypeStruct((N_NODES, D), jnp.float32),
    ],
    mesh=_MESH,
    compiler_params=_SC_PARAMS,
    scratch_types=[
        pltpu.VMEM_SHARED((N_NODES, D), jnp.float32),
        pltpu.VMEM((C,), jnp.int32),
        pltpu.VMEM((C,), jnp.int32),
        pltpu.VMEM((C,), jnp.float32),
        pltpu.VMEM((C,), jnp.float32),
        pltpu.VMEM((NSUB, SUB), jnp.int32),
        pltpu.VMEM((NSUB, SUB), jnp.int32),
        pltpu.VMEM((C, D), jnp.float32),
        pltpu.VMEM((C, D), jnp.float32),
        pltpu.SemaphoreType.DMA,
        pltpu.SemaphoreType.DMA,
        pltpu.SemaphoreType.DMA,
        pltpu.SemaphoreType.DMA,
    ],
)(_layer_body)


def _epi_body(em0, e1, e2, p3a, p3b, users, items, selo, idx, buf, gsem):
    cid = lax.axis_index("c")
    wid = lax.axis_index("s")
    w32 = cid * NS + wid

    @pl.loop(0, SEL_T)
    def _zb(j):
        buf[j] = jnp.zeros((D,), jnp.float32)

    @pl.when(w32 < NS)
    def _():
        pltpu.sync_copy(users.at[pl.ds(w32 * SEL_T, SEL_T)], idx)

    @pl.when(w32 >= NS)
    def _():
        pltpu.sync_copy(items.at[pl.ds((w32 - NS) * SEL_T, SEL_T)], idx)

    off = jnp.where(w32 < NS, 0, N_USERS).astype(jnp.int32)
    for g in range(SEL_T // D):
        idx[pl.ds(g * D, D)] = idx[pl.ds(g * D, D)] + off

    gds = [pltpu.async_copy(t.at[idx], buf, gsem, add=True)
           for t in (em0, e1, e2, p3a, p3b)]
    for dsc in gds:
        dsc.wait()

    @pl.loop(0, SEL_T)
    def _mean(r):
        buf[r] = buf[r] * 0.25

    pltpu.sync_copy(buf, selo.at[pl.ds(w32 * SEL_T, SEL_T)])


_epi = functools.partial(
    pl.kernel,
    out_type=[jax.ShapeDtypeStruct((2048, D), jnp.float32)],
    mesh=_MESH,
    compiler_params=_SC_PARAMS,
    scratch_types=[
        pltpu.VMEM((SEL_T,), jnp.int32),
        pltpu.VMEM((SEL_T, D), jnp.float32),
        pltpu.SemaphoreType.DMA,
    ],
)(_epi_body)


def _combine_body(a_ref, b_ref, o_ref):
    o_ref[...] = a_ref[...] + b_ref[...]


_combine = pl.pallas_call(
    _combine_body,
    out_shape=jax.ShapeDtypeStruct((N_NODES // D, D * D), jnp.float32),
    compiler_params=pltpu.CompilerParams(
        vmem_limit_bytes=100 * 1024 * 1024),
)


def _ratings_body(u_ref, it_ref, out_ref):
    out_ref[...] = jax.nn.sigmoid(
        jnp.dot(u_ref[...], it_ref[...], preferred_element_type=jnp.float32))


_ratings = pl.pallas_call(
    _ratings_body,
    out_shape=jax.ShapeDtypeStruct((1024, 1024), jnp.float32),
)


def _comb(pa, pb):
    e = _combine(pa.reshape(N_NODES // D, D * D),
                 pb.reshape(N_NODES // D, D * D))
    return e.reshape(N_NODES, D)


def kernel(user_table, item_table, edge_weight, edge_src, edge_dst, users, items):
    em0 = jnp.concatenate([user_table, item_table], axis=0)
    edst2 = edge_dst.reshape(N_EDGES // SUB, SUB)

    p1a, p1b = _layer(em0, edge_src, edst2, edge_weight)
    e1 = _comb(p1a, p1b)
    p2a, p2b = _layer(e1, edge_src, edst2, edge_weight)
    e2 = _comb(p2a, p2b)
    p3a, p3b = _layer(e2, edge_src, edst2, edge_weight)
    (sel,) = _epi(em0, e1, e2, p3a, p3b, users, items)
    u = sel[:1024]
    it_t = sel[1024:].T
    return _ratings(u, it_t)


# speedup vs baseline: 1.1399x; 1.0788x over previous
import functools

import jax
import jax.numpy as jnp
from jax import lax
from jax.experimental import pallas as pl
from jax.experimental.pallas import tpu as pltpu
from jax.experimental.pallas import tpu_sc as plsc

N_USERS = 50000
N_ITEMS = 50000
N_NODES = N_USERS + N_ITEMS
N_EDGES = 3200000
D = 16
NS = 16
NC = 2
NW = NC * NS
C = 640
SUB = 128
NSUB = C // SUB
NCH_C0 = 158
NCH_C1 = 158
N_CHUNKS = N_EDGES // C
FULL_ZBLK = N_NODES // C
TAIL_ROWS = N_NODES - FULL_ZBLK * C
SEL_T = 2048 // NW

_MESH = plsc.VectorSubcoreMesh(core_axis_name="c", subcore_axis_name="s",
                               num_cores=NC)
_SC_PARAMS = pltpu.CompilerParams(needs_layout_passes=False,
                                  use_tc_tiling_on_sc=False)


def _layer_body(tbl, esrc, edst2, ew, pa, pb,
                acc, src0, src1, w0, w1, dst0, dst1, rows0, rows1,
                esem, dsem, gsem, ssem):
    cid = lax.axis_index("c")
    wid = lax.axis_index("s")
    w32 = cid * NS + wid
    tile0 = jnp.where(cid == 0, wid * NCH_C0,
                      NS * NCH_C0 + wid * NCH_C1)
    nch = jnp.where(cid == 0, NCH_C0, NCH_C1)

    def gather_issue(src_v, rows_v):
        for s in range(NSUB):
            pltpu.async_copy(tbl.at[src_v.at[pl.ds(s * SUB, SUB)]],
                             rows_v.at[pl.ds(s * SUB, SUB)], gsem)

    def gather_wait(src_v, rows_v):
        pltpu.make_async_copy(tbl.at[pl.ds(0, C)], rows_v, gsem).wait()

    def scatter_issue(rows_v, dst_v):
        for s in range(NSUB):
            pltpu.async_copy(rows_v.at[pl.ds(s * SUB, SUB)],
                             acc.at[dst_v.at[s]], ssem, add=True)

    def scatter_wait(rows_v, dst_v):
        pltpu.make_async_copy(rows_v, acc.at[pl.ds(0, C)], ssem).wait()

    def srcw_issue(c, src_v, w_v):
        cc = jnp.minimum(c, N_CHUNKS - 1)
        pltpu.async_copy(esrc.at[pl.ds(cc * C, C)], src_v, esem)
        pltpu.async_copy(ew.at[pl.ds(cc * C, C)], w_v, esem)

    def srcw_wait(c, src_v, w_v):
        cc = jnp.minimum(c, N_CHUNKS - 1)
        pltpu.make_async_copy(esrc.at[pl.ds(cc * C, C)], src_v, esem).wait()
        pltpu.make_async_copy(ew.at[pl.ds(cc * C, C)], w_v, esem).wait()
        @pl.when(c >= N_CHUNKS)
        def _():
            @pl.loop(0, C // D)
            def _zw(g):
                w_v[pl.ds(g * D, D)] = jnp.zeros((D,), jnp.float32)

    def dst_issue(c, dst_v):
        cc = jnp.minimum(c, N_CHUNKS - 1)
        pltpu.async_copy(edst2.at[pl.ds(cc * NSUB, NSUB)], dst_v, dsem)

    def dst_wait(c, dst_v):
        cc = jnp.minimum(c, N_CHUNKS - 1)
        pltpu.make_async_copy(edst2.at[pl.ds(cc * NSUB, NSUB)], dst_v,
                              dsem).wait()

    def compute(rows_v, w_v):
        @pl.loop(0, C // D)
        def _g(g):
            w16 = w_v[pl.ds(g * D, D)]
            for i in range(D):
                bw = jnp.take_along_axis(
                    w16, jnp.full((D,), i, jnp.int32), axis=0,
                    mode="promise_in_bounds")
                j = g * D + i
                rows_v[j] = rows_v[j] * bw

    @pl.loop(0, C)
    def _zb(j):
        rows0[j] = jnp.zeros((D,), jnp.float32)

    for k in range(FULL_ZBLK // NS + 1):
        b = wid + NS * k

        @pl.when(b < FULL_ZBLK)
        def _():
            pltpu.sync_copy(rows0, acc.at[pl.ds(b * C, C)])

        @pl.when(b == FULL_ZBLK)
        def _():
            pltpu.sync_copy(rows0.at[pl.ds(0, TAIL_ROWS)],
                            acc.at[pl.ds(FULL_ZBLK * C, TAIL_ROWS)])
    plsc.subcore_barrier()

    pltpu.sync_copy(esrc.at[pl.ds(tile0 * C, C)], src0)
    pltpu.sync_copy(ew.at[pl.ds(tile0 * C, C)], w0)
    pltpu.sync_copy(edst2.at[pl.ds(tile0 * NSUB, NSUB)], dst0)
    gather_issue(src0, rows0)
    srcw_issue(tile0 + 1, src1, w1)

    @pl.loop(0, nch, step=2)
    def _pair(k2):
        c = tile0 + k2
        gather_wait(src0, rows0)

        @pl.when(k2 > 0)
        def _():
            scatter_wait(rows1, dst1)

        dst_issue(c + 1, dst1)
        srcw_wait(c + 1, src1, w1)
        gather_issue(src1, rows1)

        @pl.when(k2 > 0)
        def _():
            dst_wait(c, dst0)

        compute(rows0, w0)
        scatter_issue(rows0, dst0)

        @pl.when(k2 < nch - 2)
        def _():
            srcw_issue(c + 2, src0, w0)

        gather_wait(src1, rows1)
        scatter_wait(rows0, dst0)

        @pl.when(k2 < nch - 2)
        def _():
            dst_issue(c + 2, dst0)
            srcw_wait(c + 2, src0, w0)
            gather_issue(src0, rows0)

        dst_wait(c + 1, dst1)
        compute(rows1, w1)
        scatter_issue(rows1, dst1)

        @pl.when(k2 < nch - 2)
        def _():
            srcw_issue(c + 3, src1, w1)

    scatter_wait(rows1, dst1)
    plsc.subcore_barrier()

    for k in range(FULL_ZBLK // NS + 1):
        b = wid + NS * k
        for the_cid, out_t in ((0, pa), (1, pb)):

            @pl.when(jnp.logical_and(cid == the_cid, b < FULL_ZBLK))
            def _():
                pltpu.sync_copy(acc.at[pl.ds(b * C, C)],
                                out_t.at[pl.ds(b * C, C)])

            @pl.when(jnp.logical_and(cid == the_cid, b == FULL_ZBLK))
            def _():
                pltpu.sync_copy(acc.at[pl.ds(FULL_ZBLK * C, TAIL_ROWS)],
                                out_t.at[pl.ds(FULL_ZBLK * C, TAIL_ROWS)])


_layer = functools.partial(
    pl.kernel,
    out_type=[
        jax.ShapeDtypeStruct((N_NODES, D), jnp.float32),
        jax.ShapeDtypeStruct((N_NODES, D), jnp.float32),
    ],
    mesh=_MESH,
    compiler_params=_SC_PARAMS,
    scratch_types=[
        pltpu.VMEM_SHARED((N_NODES, D), jnp.float32),
        pltpu.VMEM((C,), jnp.int32),
        pltpu.VMEM((C,), jnp.int32),
        pltpu.VMEM((C,), jnp.float32),
        pltpu.VMEM((C,), jnp.float32),
        pltpu.VMEM((NSUB, SUB), jnp.int32),
        pltpu.VMEM((NSUB, SUB), jnp.int32),
        pltpu.VMEM((C, D), jnp.float32),
        pltpu.VMEM((C, D), jnp.float32),
        pltpu.SemaphoreType.DMA,
        pltpu.SemaphoreType.DMA,
        pltpu.SemaphoreType.DMA,
        pltpu.SemaphoreType.DMA,
    ],
)(_layer_body)


def _epi_body(em0, e1, e2, p3a, p3b, users, items, selo, idx, buf, gsem):
    cid = lax.axis_index("c")
    wid = lax.axis_index("s")
    w32 = cid * NS + wid

    @pl.loop(0, SEL_T)
    def _zb(j):
        buf[j] = jnp.zeros((D,), jnp.float32)

    @pl.when(w32 < NS)
    def _():
        pltpu.sync_copy(users.at[pl.ds(w32 * SEL_T, SEL_T)], idx)

    @pl.when(w32 >= NS)
    def _():
        pltpu.sync_copy(items.at[pl.ds((w32 - NS) * SEL_T, SEL_T)], idx)

    off = jnp.where(w32 < NS, 0, N_USERS).astype(jnp.int32)
    for g in range(SEL_T // D):
        idx[pl.ds(g * D, D)] = idx[pl.ds(g * D, D)] + off

    gds = [pltpu.async_copy(t.at[idx], buf, gsem, add=True)
           for t in (em0, e1, e2, p3a, p3b)]
    for dsc in gds:
        dsc.wait()

    @pl.loop(0, SEL_T)
    def _mean(r):
        buf[r] = buf[r] * 0.25

    pltpu.sync_copy(buf, selo.at[pl.ds(w32 * SEL_T, SEL_T)])


_epi = functools.partial(
    pl.kernel,
    out_type=[jax.ShapeDtypeStruct((2048, D), jnp.float32)],
    mesh=_MESH,
    compiler_params=_SC_PARAMS,
    scratch_types=[
        pltpu.VMEM((SEL_T,), jnp.int32),
        pltpu.VMEM((SEL_T, D), jnp.float32),
        pltpu.SemaphoreType.DMA,
    ],
)(_epi_body)


def _combine_body(a_ref, b_ref, o_ref):
    o_ref[...] = a_ref[...] + b_ref[...]


_combine = pl.pallas_call(
    _combine_body,
    out_shape=jax.ShapeDtypeStruct((N_NODES // D, D * D), jnp.float32),
    compiler_params=pltpu.CompilerParams(
        vmem_limit_bytes=100 * 1024 * 1024),
)


def _ratings_body(u_ref, it_ref, out_ref):
    out_ref[...] = jax.nn.sigmoid(
        jnp.dot(u_ref[...], it_ref[...], preferred_element_type=jnp.float32))


_ratings = pl.pallas_call(
    _ratings_body,
    out_shape=jax.ShapeDtypeStruct((1024, 1024), jnp.float32),
)


def _comb(pa, pb):
    e = _combine(pa.reshape(N_NODES // D, D * D),
                 pb.reshape(N_NODES // D, D * D))
    return e.reshape(N_NODES, D)


def kernel(user_table, item_table, edge_weight, edge_src, edge_dst, users, items):
    em0 = jnp.concatenate([user_table, item_table], axis=0)
    edst2 = edge_dst.reshape(N_EDGES // SUB, SUB)

    p1a, p1b = _layer(em0, edge_src, edst2, edge_weight)
    e1 = _comb(p1a, p1b)
    p2a, p2b = _layer(e1, edge_src, edst2, edge_weight)
    e2 = _comb(p2a, p2b)
    p3a, p3b = _layer(e2, edge_src, edst2, edge_weight)
    (sel,) = _epi(em0, e1, e2, p3a, p3b, users, items)
    u = sel[:1024]
    it_t = sel[1024:].T
    return _ratings(u, it_t)
